# baseline (speedup 1.0000x reference)
import functools

import jax
import jax.numpy as jnp
from jax.experimental import pallas as pl
from jax.experimental.pallas import tpu as pltpu

B, N, D, H, F = 4, 1024, 128, 8, 64
R = 256

_NEG_SLOPE = 0.2


def _adj_kernel(x_ref, xr_ref, W_ref, asrc_ref, adst_ref, out_ref):
    xb = x_ref[0]
    xr = xr_ref[0]
    W = W_ref[...]

    hi = jax.lax.Precision.HIGHEST
    u_src = jax.lax.dot_general(
        W, asrc_ref[...], (((2,), (1,)), ((0,), (0,))),
        precision=hi, preferred_element_type=jnp.float32)
    u_dst = jax.lax.dot_general(
        W, adst_ref[...], (((2,), (1,)), ((0,), (0,))),
        precision=hi, preferred_element_type=jnp.float32)

    s_rows = jax.lax.dot_general(
        xr, u_src, (((1,), (1,)), ((), ())),
        precision=hi, preferred_element_type=jnp.float32)
    d_all = jax.lax.dot_general(
        u_dst, xb, (((1,), (1,)), ((), ())),
        precision=hi, preferred_element_type=jnp.float32)

    acc = jnp.zeros((R, N), jnp.float32)
    for h in range(H):
        s = s_rows[:, h:h + 1]
        d = d_all[h:h + 1, :]
        dmax = jnp.max(d)
        sm = s + dmax
        m = jnp.maximum(sm, _NEG_SLOPE * sm)
        e1 = jnp.exp(s - m)
        e2 = jnp.exp(_NEG_SLOPE * s - m)
        f1 = jnp.exp(d)
        f2 = jnp.exp(_NEG_SLOPE * d)
        p = jnp.maximum(e1 * f1, e2 * f2)
        z = jnp.sum(p, axis=1, keepdims=True)
        acc = acc + p * ((1.0 / H) / z)
    out_ref[0] = acc


@functools.partial(jax.jit, static_argnames=())
def kernel(x, W, a_src, a_dst):
    grid = (B, N // R)
    return pl.pallas_call(
        _adj_kernel,
        grid=grid,
        in_specs=[
            pl.BlockSpec((1, N, D), lambda b, i: (b, 0, 0)),
            pl.BlockSpec((1, R, D), lambda b, i: (b, i, 0)),
            pl.BlockSpec((H, D, F), lambda b, i: (0, 0, 0)),
            pl.BlockSpec((H, F), lambda b, i: (0, 0)),
            pl.BlockSpec((H, F), lambda b, i: (0, 0)),
        ],
        out_specs=pl.BlockSpec((1, R, N), lambda b, i: (b, i, 0)),
        out_shape=jax.ShapeDtypeStruct((B, N, N), jnp.float32),
        compiler_params=pltpu.CompilerParams(
            dimension_semantics=("parallel", "parallel")),
    )(x, x, W, a_src, a_dst)


# speedup vs baseline: 1.2834x; 1.2834x over previous
import functools

import jax
import jax.numpy as jnp
from jax.experimental import pallas as pl
from jax.experimental.pallas import tpu as pltpu

B, N, D, H, F = 4, 1024, 128, 8, 64
R = 256

_NEG_SLOPE = 0.2


def _vec_kernel(x_ref, W_ref, asrc_ref, adst_ref,
                E1_ref, E2_ref, F1_ref, F2_ref):
    xb = x_ref[0]
    W = W_ref[...]
    hi = jax.lax.Precision.HIGHEST
    u_src = jax.lax.dot_general(
        W, asrc_ref[...], (((2,), (1,)), ((0,), (0,))),
        precision=hi, preferred_element_type=jnp.float32)
    u_dst = jax.lax.dot_general(
        W, adst_ref[...], (((2,), (1,)), ((0,), (0,))),
        precision=hi, preferred_element_type=jnp.float32)
    s = jax.lax.dot_general(
        xb, u_src, (((1,), (1,)), ((), ())),
        precision=hi, preferred_element_type=jnp.float32)
    dT = jax.lax.dot_general(
        u_dst, xb, (((1,), (1,)), ((), ())),
        precision=hi, preferred_element_type=jnp.float32)
    dmax = jnp.max(dT, axis=1)
    sm = s + dmax[None, :]
    m = jnp.maximum(sm, _NEG_SLOPE * sm)
    E1_ref[0] = jnp.exp(s - m)
    E2_ref[0] = jnp.exp(_NEG_SLOPE * s - m)
    F1_ref[0] = jnp.exp(dT)
    F2_ref[0] = jnp.exp(_NEG_SLOPE * dT)


def _adj_kernel(E1_ref, E2_ref, F1_ref, F2_ref, out_ref):
    acc = jnp.zeros((R, N), jnp.float32)
    for h in range(H):
        e1 = E1_ref[0][:, h:h + 1]
        e2 = E2_ref[0][:, h:h + 1]
        f1 = F1_ref[0][h:h + 1, :]
        f2 = F2_ref[0][h:h + 1, :]
        p = jnp.maximum(e1 * f1, e2 * f2)
        z = jnp.sum(p, axis=1, keepdims=True)
        acc = acc + p * ((1.0 / H) / z)
    out_ref[0] = acc


@functools.partial(jax.jit, static_argnames=())
def kernel(x, W, a_src, a_dst):
    vec_shape = jax.ShapeDtypeStruct((B, N, H), jnp.float32)
    vecT_shape = jax.ShapeDtypeStruct((B, H, N), jnp.float32)
    E1, E2, F1, F2 = pl.pallas_call(
        _vec_kernel,
        grid=(B,),
        in_specs=[
            pl.BlockSpec((1, N, D), lambda b: (b, 0, 0)),
            pl.BlockSpec((H, D, F), lambda b: (0, 0, 0)),
            pl.BlockSpec((H, F), lambda b: (0, 0)),
            pl.BlockSpec((H, F), lambda b: (0, 0)),
        ],
        out_specs=[
            pl.BlockSpec((1, N, H), lambda b: (b, 0, 0)),
            pl.BlockSpec((1, N, H), lambda b: (b, 0, 0)),
            pl.BlockSpec((1, H, N), lambda b: (b, 0, 0)),
            pl.BlockSpec((1, H, N), lambda b: (b, 0, 0)),
        ],
        out_shape=[vec_shape, vec_shape, vecT_shape, vecT_shape],
    )(x, W, a_src, a_dst)

    return pl.pallas_call(
        _adj_kernel,
        grid=(B, N // R),
        in_specs=[
            pl.BlockSpec((1, R, H), lambda b, i: (b, i, 0)),
            pl.BlockSpec((1, R, H), lambda b, i: (b, i, 0)),
            pl.BlockSpec((1, H, N), lambda b, i: (b, 0, 0)),
            pl.BlockSpec((1, H, N), lambda b, i: (b, 0, 0)),
        ],
        out_specs=pl.BlockSpec((1, R, N), lambda b, i: (b, i, 0)),
        out_shape=jax.ShapeDtypeStruct((B, N, N), jnp.float32),
        compiler_params=pltpu.CompilerParams(
            dimension_semantics=("parallel", "parallel")),
    )(E1, E2, F1, F2)
